# baseline (speedup 1.0000x reference)
import jax
import jax.numpy as jnp
from jax.experimental import pallas as pl
from jax.experimental.pallas import tpu as pltpu

C = 1024
K = 4096
TOPK = 20
T = 128
NSLICE = 16
W = K // NSLICE
DEPTH = 4
NEG = float("-inf")


def _row_kth_full(zpre):
    w = zpre
    for _ in range(TOPK - 1):
        m = jnp.max(w, axis=1, keepdims=True)
        w = jnp.where(w == m, NEG, w)
    return jnp.max(w, axis=1, keepdims=True)


def _row_kth_hier(zpre):
    slices = [zpre[:, i * W:(i + 1) * W] for i in range(NSLICE)]

    neg = jnp.full(slices[0].shape, NEG, jnp.float32)
    regs = [slices[0]] + [neg] * (DEPTH - 1)
    for s in slices[1:]:
        cand = s
        for d in range(DEPTH):
            hi = jnp.maximum(regs[d], cand)
            if d < DEPTH - 1:
                cand = jnp.minimum(regs[d], cand)
            regs[d] = hi

    cur, nn = regs[0], regs[1:]
    for _ in range(TOPK - 1):
        m = jnp.max(cur, axis=1, keepdims=True)
        sel = cur == m
        cur = jnp.where(sel, nn[0], cur)
        for d in range(len(nn) - 1):
            nn[d] = jnp.where(sel, nn[d + 1], nn[d])
        nn[-1] = jnp.where(sel, NEG, nn[-1])
    of = jnp.any(cur == NEG)
    thresh_fast = jnp.max(cur, axis=1, keepdims=True)
    return jax.lax.cond(of, lambda: _row_kth_full(zpre), lambda: thresh_fast)


def _body(x_ref, ew_ref, eb_ref, dw_ref, db_ref, z_ref, xh_ref):
    xb = x_ref[...]
    ew = ew_ref[...]
    zpre = jax.lax.dot_general(
        xb, ew, (((1,), (1,)), ((), ())),
        preferred_element_type=jnp.float32)
    zpre = zpre + eb_ref[...]

    thresh = _row_kth_hier(zpre)

    t2 = jnp.maximum(thresh, 0.0)
    z = jnp.where(zpre >= t2, zpre, 0.0)
    z_ref[...] = z

    dw = dw_ref[...]
    xh = jax.lax.dot_general(
        z.astype(jnp.bfloat16), dw, (((1,), (1,)), ((), ())),
        preferred_element_type=jnp.float32)
    xh_ref[...] = xh + db_ref[...]


def kernel(x, enc_w, enc_b, dec_w, dec_b):
    B, N, _ = x.shape
    M = B * N
    xf = x.reshape(M, C).astype(jnp.bfloat16)
    ew = enc_w.astype(jnp.bfloat16)
    dw = dec_w.astype(jnp.bfloat16)
    eb = enc_b.reshape(1, K)
    db = dec_b.reshape(1, C)

    z, xh = pl.pallas_call(
        _body,
        grid=(M // T,),
        in_specs=[
            pl.BlockSpec((T, C), lambda i: (i, 0)),
            pl.BlockSpec((K, C), lambda i: (0, 0)),
            pl.BlockSpec((1, K), lambda i: (0, 0)),
            pl.BlockSpec((C, K), lambda i: (0, 0)),
            pl.BlockSpec((1, C), lambda i: (0, 0)),
        ],
        out_specs=[
            pl.BlockSpec((T, K), lambda i: (i, 0)),
            pl.BlockSpec((T, C), lambda i: (i, 0)),
        ],
        out_shape=[
            jax.ShapeDtypeStruct((M, K), jnp.float32),
            jax.ShapeDtypeStruct((M, C), jnp.float32),
        ],
        compiler_params=pltpu.CompilerParams(
            dimension_semantics=("parallel",)),
    )(xf, ew, eb, dw, db)
    return z.reshape(B, N, K), xh.reshape(B, N, C)


# speedup vs baseline: 1.2032x; 1.2032x over previous
import jax
import jax.numpy as jnp
from jax.experimental import pallas as pl
from jax.experimental.pallas import tpu as pltpu

C = 1024
K = 4096
TOPK = 20
T = 256
H = T // 2
NSLICE = 16
W = K // NSLICE
DEPTH = 4
NEG = float("-inf")


def _row_kth_full(zpre):
    w = zpre
    for _ in range(TOPK - 1):
        m = jnp.max(w, axis=1, keepdims=True)
        w = jnp.where(w == m, NEG, w)
    return jnp.max(w, axis=1, keepdims=True)


def _row_kth_hier(zpre):
    slices = [zpre[:, i * W:(i + 1) * W] for i in range(NSLICE)]

    neg = jnp.full(slices[0].shape, NEG, jnp.float32)
    regs = [slices[0]] + [neg] * (DEPTH - 1)
    for s in slices[1:]:
        cand = s
        for d in range(DEPTH):
            hi = jnp.maximum(regs[d], cand)
            if d < DEPTH - 1:
                cand = jnp.minimum(regs[d], cand)
            regs[d] = hi

    cur, nn = regs[0], regs[1:]
    for _ in range(TOPK - 1):
        m = jnp.max(cur, axis=1, keepdims=True)
        sel = cur == m
        cur = jnp.where(sel, nn[0], cur)
        for d in range(len(nn) - 1):
            nn[d] = jnp.where(sel, nn[d + 1], nn[d])
        nn[-1] = jnp.where(sel, NEG, nn[-1])
    of = jnp.any(cur == NEG)
    thresh_fast = jnp.max(cur, axis=1, keepdims=True)
    return jax.lax.cond(of, lambda: _row_kth_full(zpre), lambda: thresh_fast)


def _body(x_ref, ew_ref, eb_ref, dw_ref, db_ref, z_ref, xh_ref):
    xb = x_ref[...]
    ew = ew_ref[...]
    dw = dw_ref[...]

    zpre = [
        jax.lax.dot_general(
            xb[h * H:(h + 1) * H], ew, (((1,), (1,)), ((), ())),
            preferred_element_type=jnp.float32) + eb_ref[...]
        for h in range(2)
    ]

    for h in range(2):
        zp = zpre[h]
        thresh = _row_kth_hier(zp)
        t2 = jnp.maximum(thresh, 0.0)
        z = jnp.where(zp >= t2, zp, 0.0)
        z_ref[h * H:(h + 1) * H, :] = z
        xhat = jax.lax.dot_general(
            z.astype(jnp.bfloat16), dw, (((1,), (1,)), ((), ())),
            preferred_element_type=jnp.float32)
        xh_ref[h * H:(h + 1) * H, :] = xhat + db_ref[...]


def kernel(x, enc_w, enc_b, dec_w, dec_b):
    B, N, _ = x.shape
    M = B * N
    xf = x.reshape(M, C).astype(jnp.bfloat16)
    ew = enc_w.astype(jnp.bfloat16)
    dw = dec_w.astype(jnp.bfloat16)
    eb = enc_b.reshape(1, K)
    db = dec_b.reshape(1, C)

    z, xh = pl.pallas_call(
        _body,
        grid=(M // T,),
        in_specs=[
            pl.BlockSpec((T, C), lambda i: (i, 0)),
            pl.BlockSpec((K, C), lambda i: (0, 0)),
            pl.BlockSpec((1, K), lambda i: (0, 0)),
            pl.BlockSpec((C, K), lambda i: (0, 0)),
            pl.BlockSpec((1, C), lambda i: (0, 0)),
        ],
        out_specs=[
            pl.BlockSpec((T, K), lambda i: (i, 0)),
            pl.BlockSpec((T, C), lambda i: (i, 0)),
        ],
        out_shape=[
            jax.ShapeDtypeStruct((M, K), jnp.float32),
            jax.ShapeDtypeStruct((M, C), jnp.float32),
        ],
        compiler_params=pltpu.CompilerParams(
            dimension_semantics=("parallel",)),
    )(xf, ew, eb, dw, db)
    return z.reshape(B, N, K), xh.reshape(B, N, C)


# speedup vs baseline: 1.9189x; 1.5948x over previous
import jax
import jax.numpy as jnp
from jax.experimental import pallas as pl
from jax.experimental.pallas import tpu as pltpu

C = 1024
K = 4096
TOPK = 20
T = 256
NSLICE = 16
W = K // NSLICE
DEPTH = 4
NEG = float("-inf")


def _row_kth_full(zpre):
    w = zpre
    for _ in range(TOPK - 1):
        m = jnp.max(w, axis=1, keepdims=True)
        w = jnp.where(w == m, NEG, w)
    return jnp.max(w, axis=1, keepdims=True)


def _row_kth_hier(zpre):
    slices = [zpre[:, i * W:(i + 1) * W] for i in range(NSLICE)]

    neg = jnp.full(slices[0].shape, NEG, jnp.float32)
    regs = [slices[0]] + [neg] * (DEPTH - 1)
    for s in slices[1:]:
        cand = s
        for d in range(DEPTH):
            hi = jnp.maximum(regs[d], cand)
            if d < DEPTH - 1:
                cand = jnp.minimum(regs[d], cand)
            regs[d] = hi

    cur, nn = regs[0], regs[1:]
    for _ in range(TOPK - 1):
        m = jnp.max(cur, axis=1, keepdims=True)
        sel = cur == m
        cur = jnp.where(sel, nn[0], cur)
        for d in range(len(nn) - 1):
            nn[d] = jnp.where(sel, nn[d + 1], nn[d])
        nn[-1] = jnp.where(sel, NEG, nn[-1])
    of = jnp.any(cur == NEG)
    thresh_fast = jnp.max(cur, axis=1, keepdims=True)
    return jax.lax.cond(of, lambda: _row_kth_full(zpre), lambda: thresh_fast)


def _body(x_ref, ew_ref, eb_ref, dw_ref, db_ref, z_ref, xh_ref):
    xb = x_ref[...]
    ew = ew_ref[...]
    zpre = jax.lax.dot_general(
        xb, ew, (((1,), (1,)), ((), ())),
        preferred_element_type=jnp.float32)
    zpre = zpre + eb_ref[...]

    thresh = _row_kth_hier(zpre)

    t2 = jnp.maximum(thresh, 0.0)
    z = jnp.where(zpre >= t2, zpre, 0.0)
    z_ref[...] = z

    dw = dw_ref[...]
    xh = jax.lax.dot_general(
        z.astype(jnp.bfloat16), dw, (((1,), (1,)), ((), ())),
        preferred_element_type=jnp.float32)
    xh_ref[...] = xh + db_ref[...]


def kernel(x, enc_w, enc_b, dec_w, dec_b):
    B, N, _ = x.shape
    M = B * N
    xf = x.reshape(M, C).astype(jnp.bfloat16)
    ew = enc_w.astype(jnp.bfloat16)
    dw = dec_w.astype(jnp.bfloat16)
    eb = enc_b.reshape(1, K)
    db = dec_b.reshape(1, C)

    z, xh = pl.pallas_call(
        _body,
        grid=(M // T,),
        in_specs=[
            pl.BlockSpec((T, C), lambda i: (i, 0)),
            pl.BlockSpec((K, C), lambda i: (0, 0)),
            pl.BlockSpec((1, K), lambda i: (0, 0)),
            pl.BlockSpec((C, K), lambda i: (0, 0)),
            pl.BlockSpec((1, C), lambda i: (0, 0)),
        ],
        out_specs=[
            pl.BlockSpec((T, K), lambda i: (i, 0)),
            pl.BlockSpec((T, C), lambda i: (i, 0)),
        ],
        out_shape=[
            jax.ShapeDtypeStruct((M, K), jnp.float32),
            jax.ShapeDtypeStruct((M, C), jnp.float32),
        ],
        compiler_params=pltpu.CompilerParams(
            dimension_semantics=("arbitrary",)),
    )(xf, ew, eb, dw, db)
    return z.reshape(B, N, K), xh.reshape(B, N, C)


# speedup vs baseline: 1.9271x; 1.0042x over previous
import jax
import jax.numpy as jnp
from jax.experimental import pallas as pl
from jax.experimental.pallas import tpu as pltpu

C = 1024
K = 4096
TOPK = 20
T = 256
NSLICE = 16
W = K // NSLICE
DEPTH = 4
NEG = float("-inf")


def _row_kth_full(zpre):
    w = zpre
    for _ in range(TOPK - 1):
        m = jnp.max(w, axis=1, keepdims=True)
        w = jnp.where(w == m, NEG, w)
    return jnp.max(w, axis=1, keepdims=True)


def _row_kth_hier(zpre):
    slices = [zpre[:, i * W:(i + 1) * W] for i in range(NSLICE)]

    neg = jnp.full(slices[0].shape, NEG, jnp.float32)
    regs = [slices[0]] + [neg] * (DEPTH - 1)
    for s in slices[1:]:
        cand = s
        for d in range(DEPTH):
            hi = jnp.maximum(regs[d], cand)
            if d < DEPTH - 1:
                cand = jnp.minimum(regs[d], cand)
            regs[d] = hi

    cur, nn = regs[0], regs[1:]
    for _ in range(TOPK - 1):
        m = jnp.max(cur, axis=1, keepdims=True)
        sel = cur == m
        cur = jnp.where(sel, nn[0], cur)
        for d in range(len(nn) - 1):
            nn[d] = jnp.where(sel, nn[d + 1], nn[d])
        nn[-1] = jnp.where(sel, NEG, nn[-1])
    of = jnp.any(cur == NEG)
    thresh_fast = jnp.max(cur, axis=1, keepdims=True)
    return jax.lax.cond(of, lambda: _row_kth_full(zpre), lambda: thresh_fast)


def _body(x_ref, ew_ref, eb_ref, dw_ref, db_ref, z_ref, xh_ref):
    xb = x_ref[...]
    ew = ew_ref[...]
    zpre = jax.lax.dot_general(
        xb, ew, (((1,), (1,)), ((), ())),
        preferred_element_type=jnp.float32)
    zpre = zpre + eb_ref[...]

    thresh = _row_kth_hier(zpre)

    t2 = jnp.maximum(thresh, 0.0)
    z = jnp.where(zpre >= t2, zpre, 0.0)
    z_ref[...] = z

    dw = dw_ref[...]
    xh = jax.lax.dot_general(
        z.astype(jnp.bfloat16), dw, (((1,), (1,)), ((), ())),
        preferred_element_type=jnp.float32)
    xh_ref[...] = xh + db_ref[...]


def kernel(x, enc_w, enc_b, dec_w, dec_b):
    B, N, _ = x.shape
    M = B * N
    xf = x.reshape(M, C).astype(jnp.bfloat16)
    ew = enc_w.astype(jnp.bfloat16)
    dw = dec_w.astype(jnp.bfloat16)
    eb = enc_b.reshape(1, K)
    db = dec_b.reshape(1, C)

    z, xh = pl.pallas_call(
        _body,
        grid=(M // T,),
        in_specs=[
            pl.BlockSpec((T, C), lambda i: (i, 0)),
            pl.BlockSpec((K, C), lambda i: (0, 0)),
            pl.BlockSpec((1, K), lambda i: (0, 0)),
            pl.BlockSpec((C, K), lambda i: (0, 0)),
            pl.BlockSpec((1, C), lambda i: (0, 0)),
        ],
        out_specs=[
            pl.BlockSpec((T, K), lambda i: (i, 0)),
            pl.BlockSpec((T, C), lambda i: (i, 0)),
        ],
        out_shape=[
            jax.ShapeDtypeStruct((M, K), jnp.float32),
            jax.ShapeDtypeStruct((M, C), jnp.float32),
        ],
        compiler_params=pltpu.CompilerParams(
            dimension_semantics=("parallel",)),
    )(xf, ew, eb, dw, db)
    return z.reshape(B, N, K), xh.reshape(B, N, C)
